# baseline (speedup 1.0000x reference)
import jax
import jax.numpy as jnp
from jax.experimental import pallas as pl
from jax.experimental.pallas import tpu as pltpu

_N = 20000
_PAD_N = 32768
_ROWS = 256
_LANES = 128
_TOP_ROWS = 48
_PRE_TOPK = 6000
_POST_TOPK = 1000
_THRESH = 0.7


def _nms_body(sc_in, x1_in, y1_in, x2_in, y2_in, out_ref,
              ss_ref, si_ref, sx1_ref, sy1_ref, sx2_ref, sy2_ref,
              ar_ref, msk_ref):
    i32 = jnp.int32
    f32 = jnp.float32
    neg_inf = jnp.float32(-jnp.inf)

    row_iota = jax.lax.broadcasted_iota(i32, (_ROWS, _LANES), 0)
    lane_iota = jax.lax.broadcasted_iota(i32, (_ROWS, _LANES), 1)
    flat_iota = row_iota * _LANES + lane_iota
    lane1 = jax.lax.broadcasted_iota(i32, (1, _LANES), 1)

    out_ref[:] = jnp.full((_POST_TOPK, _LANES), -1.0, f32)
    ss_ref[:] = sc_in[:]
    si_ref[:] = flat_iota
    sx1_ref[:] = x1_in[:]
    sy1_ref[:] = y1_in[:]
    sx2_ref[:] = x2_in[:]
    sy2_ref[:] = y2_in[:]

    def ce(d, k, axis, sp, sm):
        bit = (flat_iota & d) != 0
        dsc = (flat_iota & k) != 0

        def partner(x):
            return jnp.where(bit, pltpu.roll(x, sp, axis), pltpu.roll(x, sm, axis))

        s = ss_ref[:]
        ii = si_ref[:]
        ps = partner(s)
        pi = partner(ii)
        first = (s > ps) | ((s == ps) & (ii < pi))
        keep = first ^ bit ^ dsc
        ss_ref[:] = jnp.where(keep, s, ps)
        si_ref[:] = jnp.where(keep, ii, pi)
        for ref in (sx1_ref, sy1_ref, sx2_ref, sy2_ref):
            x = ref[:]
            ref[:] = jnp.where(keep, x, partner(x))

    def row_step(t, j):
        d = jax.lax.shift_left(jnp.int32(1), j - 1 - t)
        k = jax.lax.shift_left(jnp.int32(1), j)
        dr = d >> 7
        ce(d, k, 0, dr, (_ROWS - dr) & (_ROWS - 1))
        return j

    def lane_step(t, j):
        d = jax.lax.shift_left(jnp.int32(1), j - 1 - t)
        k = jax.lax.shift_left(jnp.int32(1), j)
        ce(d, k, 1, d, (_LANES - d) & (_LANES - 1))
        return j

    def stage(j, _):
        split = jnp.maximum(j - 7, 0)
        jax.lax.fori_loop(0, split, row_step, j)
        jax.lax.fori_loop(split, j, lane_step, j)
        return 0

    jax.lax.fori_loop(1, 16, stage, 0)

    top_flat = flat_iota[:_TOP_ROWS, :]
    lane_top = lane_iota[:_TOP_ROWS, :]

    X1 = sx1_ref[0:_TOP_ROWS, :]
    Y1 = sy1_ref[0:_TOP_ROWS, :]
    X2 = sx2_ref[0:_TOP_ROWS, :]
    Y2 = sy2_ref[0:_TOP_ROWS, :]
    ar_ref[:] = (X2 - X1) * (Y2 - Y1)
    msk_ref[:] = jnp.where(top_flat < _PRE_TOPK, ss_ref[0:_TOP_ROWS, :], neg_inf)

    big_i = jnp.int32(0x7FFFFFFF)

    def find_next(msk_val):
        cand = jnp.where(msk_val > neg_inf, top_flat, big_i)
        a = jnp.minimum(cand[0:16, :], cand[16:32, :])
        a = jnp.minimum(a, cand[32:48, :])
        a = jnp.minimum(a[0:8, :], a[8:16, :])
        a = jnp.minimum(a, pltpu.roll(a, 4, 0))
        a = jnp.minimum(a, pltpu.roll(a, 2, 0))
        a = jnp.minimum(a, pltpu.roll(a, 1, 0))
        t = a[0:1, :]
        for s in (64, 32, 16, 8, 4, 2, 1):
            t = jnp.minimum(t, pltpu.roll(t, s, 1))
        return jnp.min(t[:, 0:1])

    def cond(carry):
        b, cnt = carry
        return (b < big_i) & (cnt < _POST_TOPK)

    def body(carry):
        b, cnt = carry
        r = b >> 7
        c = b & 127
        shift = (_LANES - c) & (_LANES - 1)

        def ext(ref):
            rolled = pltpu.roll(ref[pl.ds(r, 1), :], shift, 1)
            return jnp.broadcast_to(rolled[:, 0:1], (1, _LANES))

        bx1 = ext(sx1_ref)
        by1 = ext(sy1_ref)
        bx2 = ext(sx2_ref)
        by2 = ext(sy2_ref)
        bsc = ext(msk_ref)

        ix1 = jnp.maximum(bx1, sx1_ref[0:_TOP_ROWS, :])
        iy1 = jnp.maximum(by1, sy1_ref[0:_TOP_ROWS, :])
        ix2 = jnp.minimum(bx2, sx2_ref[0:_TOP_ROWS, :])
        iy2 = jnp.minimum(by2, sy2_ref[0:_TOP_ROWS, :])
        inter = jnp.maximum(ix2 - ix1, 0.0) * jnp.maximum(iy2 - iy1, 0.0)
        area_a = (bx2 - bx1) * (by2 - by1)
        iou = inter / (area_a + ar_ref[:] - inter + 1e-9)
        keep = (iou < _THRESH) & (top_flat != b)
        new_msk = jnp.where(keep, msk_ref[:], neg_inf)
        msk_ref[:] = new_msk

        row = jnp.where(
            lane1 == 0, bx1,
            jnp.where(lane1 == 1, by1,
                      jnp.where(lane1 == 2, bx2,
                                jnp.where(lane1 == 3, by2,
                                          jnp.where(lane1 == 4, bsc, -1.0)))))
        out_ref[pl.ds(cnt, 1), :] = row

        return find_next(new_msk), cnt + 1

    jax.lax.while_loop(cond, body, (find_next(msk_ref[:]), jnp.int32(0)))


def _pad2d(v, fill):
    v = jnp.concatenate([v, jnp.full((_PAD_N - _N,), fill, jnp.float32)])
    return v.reshape(_ROWS, _LANES)


def kernel(boxes, scores):
    sc = _pad2d(scores, -1.0)
    x1 = _pad2d(boxes[:, 0], 0.0)
    y1 = _pad2d(boxes[:, 1], 0.0)
    x2 = _pad2d(boxes[:, 2], 0.0)
    y2 = _pad2d(boxes[:, 3], 0.0)

    big = pltpu.VMEM((_ROWS, _LANES), jnp.float32)
    top = pltpu.VMEM((_TOP_ROWS, _LANES), jnp.float32)
    out = pl.pallas_call(
        _nms_body,
        out_shape=jax.ShapeDtypeStruct((_POST_TOPK, _LANES), jnp.float32),
        scratch_shapes=[
            big, pltpu.VMEM((_ROWS, _LANES), jnp.int32),
            big, big, big, big,
            top, top,
        ],
    )(sc, x1, y1, x2, y2)
    return out[:, :5]


# speedup vs baseline: 2.5318x; 2.5318x over previous
import jax
import jax.numpy as jnp
from jax.experimental import pallas as pl
from jax.experimental.pallas import tpu as pltpu

_N = 20000
_PAD_N = 32768
_ROWS = 256
_LANES = 128
_TOP_ROWS = 48
_PRE_TOPK = 6000
_POST_TOPK = 1000
_THRESH = 0.7


def _nms_body(sc_in, x1_in, y1_in, x2_in, y2_in, out_ref,
              ss_ref, si_ref, sx1_ref, sy1_ref, sx2_ref, sy2_ref,
              ar_ref, msk_ref, grp_ref):
    i32 = jnp.int32
    f32 = jnp.float32
    neg_inf = jnp.float32(-jnp.inf)

    row_iota = jax.lax.broadcasted_iota(i32, (_ROWS, _LANES), 0)
    lane_iota = jax.lax.broadcasted_iota(i32, (_ROWS, _LANES), 1)
    flat_iota = row_iota * _LANES + lane_iota
    lane1 = jax.lax.broadcasted_iota(i32, (1, _LANES), 1)

    out_ref[:] = jnp.full((_POST_TOPK, _LANES), -1.0, f32)
    ss_ref[:] = sc_in[:]
    si_ref[:] = flat_iota
    sx1_ref[:] = x1_in[:]
    sy1_ref[:] = y1_in[:]
    sx2_ref[:] = x2_in[:]
    sy2_ref[:] = y2_in[:]

    def ce(d, k, axis, sp, sm):
        bit = (flat_iota & d) != 0
        dsc = (flat_iota & k) != 0

        def partner(x):
            return jnp.where(bit, pltpu.roll(x, sp, axis), pltpu.roll(x, sm, axis))

        s = ss_ref[:]
        ii = si_ref[:]
        ps = partner(s)
        pi = partner(ii)
        first = (s > ps) | ((s == ps) & (ii < pi))
        keep = first ^ bit ^ dsc
        ss_ref[:] = jnp.where(keep, s, ps)
        si_ref[:] = jnp.where(keep, ii, pi)
        for ref in (sx1_ref, sy1_ref, sx2_ref, sy2_ref):
            x = ref[:]
            ref[:] = jnp.where(keep, x, partner(x))

    def row_step(t, j):
        d = jax.lax.shift_left(jnp.int32(1), j - 1 - t)
        k = jax.lax.shift_left(jnp.int32(1), j)
        dr = d >> 7
        ce(d, k, 0, dr, (_ROWS - dr) & (_ROWS - 1))
        return j

    def lane_step(t, j):
        d = jax.lax.shift_left(jnp.int32(1), j - 1 - t)
        k = jax.lax.shift_left(jnp.int32(1), j)
        ce(d, k, 1, d, (_LANES - d) & (_LANES - 1))
        return j

    def stage(j, _):
        split = jnp.maximum(j - 7, 0)
        jax.lax.fori_loop(0, split, row_step, j)
        jax.lax.fori_loop(split, j, lane_step, j)
        return 0

    jax.lax.fori_loop(1, 16, stage, 0)

    top_flat = flat_iota[:_TOP_ROWS, :]
    lane_top = lane_iota[:_TOP_ROWS, :]

    X1 = sx1_ref[0:_TOP_ROWS, :]
    Y1 = sy1_ref[0:_TOP_ROWS, :]
    X2 = sx2_ref[0:_TOP_ROWS, :]
    Y2 = sy2_ref[0:_TOP_ROWS, :]
    ar_ref[:] = (X2 - X1) * (Y2 - Y1)
    msk_ref[:] = (top_flat < _PRE_TOPK).astype(f32)

    def fill_grp(r, _):
        grp = jnp.concatenate(
            [sx1_ref[pl.ds(r, 1), :], sy1_ref[pl.ds(r, 1), :],
             sx2_ref[pl.ds(r, 1), :], sy2_ref[pl.ds(r, 1), :],
             ss_ref[pl.ds(r, 1), :],
             jnp.zeros((3, _LANES), f32)], axis=0)
        grp_ref[pl.ds(r * 8, 8), :] = grp
        return 0

    jax.lax.fori_loop(0, _TOP_ROWS, fill_grp, 0)

    def cond(carry):
        j, cnt = carry
        return (j < _TOP_ROWS * _LANES) & (cnt < _POST_TOPK)

    def body(carry):
        j, cnt = carry
        r = j >> 7
        c = j & 127
        onehot = (lane1 == c).astype(f32)

        vrow = msk_ref[pl.ds(r, 1), :]
        isv = jnp.max(jax.lax.dot_general(
            vrow, onehot, (((1,), (1,)), ((), ())),
            preferred_element_type=f32)) > 0.5

        @pl.when(isv)
        def _():
            grp = grp_ref[pl.ds(r * 8, 8), :]
            col = jax.lax.dot_general(
                grp, onehot, (((1,), (1,)), ((), ())),
                preferred_element_type=f32)

            def bc(k):
                return jnp.broadcast_to(col[k:k + 1, :], (1, _LANES))

            bx1 = bc(0)
            by1 = bc(1)
            bx2 = bc(2)
            by2 = bc(3)
            bsc = bc(4)

            ix1 = jnp.maximum(bx1, sx1_ref[0:_TOP_ROWS, :])
            iy1 = jnp.maximum(by1, sy1_ref[0:_TOP_ROWS, :])
            ix2 = jnp.minimum(bx2, sx2_ref[0:_TOP_ROWS, :])
            iy2 = jnp.minimum(by2, sy2_ref[0:_TOP_ROWS, :])
            inter = jnp.maximum(ix2 - ix1, 0.0) * jnp.maximum(iy2 - iy1, 0.0)
            area_a = (bx2 - bx1) * (by2 - by1)
            iou = inter / (area_a + ar_ref[:] - inter + 1e-9)
            keep = (iou < _THRESH) & (top_flat != j)
            msk_ref[:] = jnp.where(keep, msk_ref[:], 0.0)

            row = jnp.where(
                lane1 == 0, bx1,
                jnp.where(lane1 == 1, by1,
                          jnp.where(lane1 == 2, bx2,
                                    jnp.where(lane1 == 3, by2,
                                              jnp.where(lane1 == 4, bsc, -1.0)))))
            out_ref[pl.ds(cnt, 1), :] = row

        return j + 1, cnt + isv.astype(i32)

    jax.lax.while_loop(cond, body, (jnp.int32(0), jnp.int32(0)))


def _pad2d(v, fill):
    v = jnp.concatenate([v, jnp.full((_PAD_N - _N,), fill, jnp.float32)])
    return v.reshape(_ROWS, _LANES)


def kernel(boxes, scores):
    sc = _pad2d(scores, -1.0)
    x1 = _pad2d(boxes[:, 0], 0.0)
    y1 = _pad2d(boxes[:, 1], 0.0)
    x2 = _pad2d(boxes[:, 2], 0.0)
    y2 = _pad2d(boxes[:, 3], 0.0)

    big = pltpu.VMEM((_ROWS, _LANES), jnp.float32)
    top = pltpu.VMEM((_TOP_ROWS, _LANES), jnp.float32)
    out = pl.pallas_call(
        _nms_body,
        out_shape=jax.ShapeDtypeStruct((_POST_TOPK, _LANES), jnp.float32),
        scratch_shapes=[
            big, pltpu.VMEM((_ROWS, _LANES), jnp.int32),
            big, big, big, big,
            top, top,
            pltpu.VMEM((_TOP_ROWS * 8, _LANES), jnp.float32),
        ],
    )(sc, x1, y1, x2, y2)
    return out[:, :5]


# speedup vs baseline: 3.3243x; 1.3130x over previous
import jax
import jax.numpy as jnp
from jax.experimental import pallas as pl
from jax.experimental.pallas import tpu as pltpu

_N = 20000
_PAD_N = 32768
_ROWS = 256
_LANES = 128
_TOP_ROWS = 48
_PRE_TOPK = 6000
_POST_TOPK = 1000
_THRESH = 0.7


def _nms_body(sc_in, x1_in, y1_in, x2_in, y2_in, out_ref,
              ss_ref, si_ref, sx1_ref, sy1_ref, sx2_ref, sy2_ref,
              ar_ref, msk_ref, grp_ref):
    i32 = jnp.int32
    f32 = jnp.float32
    neg_inf = jnp.float32(-jnp.inf)

    row_iota = jax.lax.broadcasted_iota(i32, (_ROWS, _LANES), 0)
    lane_iota = jax.lax.broadcasted_iota(i32, (_ROWS, _LANES), 1)
    flat_iota = row_iota * _LANES + lane_iota
    lane1 = jax.lax.broadcasted_iota(i32, (1, _LANES), 1)

    out_ref[:] = jnp.full((_POST_TOPK, _LANES), -1.0, f32)
    ss_ref[:] = sc_in[:]
    si_ref[:] = flat_iota
    sx1_ref[:] = x1_in[:]
    sy1_ref[:] = y1_in[:]
    sx2_ref[:] = x2_in[:]
    sy2_ref[:] = y2_in[:]

    def ce(d, k, axis, sp, sm):
        bit = (flat_iota & d) != 0
        dsc = (flat_iota & k) != 0

        def partner(x):
            return jnp.where(bit, pltpu.roll(x, sp, axis), pltpu.roll(x, sm, axis))

        s = ss_ref[:]
        ii = si_ref[:]
        ps = partner(s)
        pi = partner(ii)
        first = (s > ps) | ((s == ps) & (ii < pi))
        keep = first ^ bit ^ dsc
        ss_ref[:] = jnp.where(keep, s, ps)
        si_ref[:] = jnp.where(keep, ii, pi)
        for ref in (sx1_ref, sy1_ref, sx2_ref, sy2_ref):
            x = ref[:]
            ref[:] = jnp.where(keep, x, partner(x))

    def row_step(t, j):
        d = jax.lax.shift_left(jnp.int32(1), j - 1 - t)
        k = jax.lax.shift_left(jnp.int32(1), j)
        dr = d >> 7
        ce(d, k, 0, dr, (_ROWS - dr) & (_ROWS - 1))
        return j

    def lane_step(t, j):
        d = jax.lax.shift_left(jnp.int32(1), j - 1 - t)
        k = jax.lax.shift_left(jnp.int32(1), j)
        ce(d, k, 1, d, (_LANES - d) & (_LANES - 1))
        return j

    def stage(j, _):
        split = jnp.maximum(j - 7, 0)
        jax.lax.fori_loop(0, split, row_step, j)
        jax.lax.fori_loop(split, j, lane_step, j)
        return 0

    jax.lax.fori_loop(1, 16, stage, 0)

    top_flat = flat_iota[:_TOP_ROWS, :]
    lane_top = lane_iota[:_TOP_ROWS, :]

    X1 = sx1_ref[0:_TOP_ROWS, :]
    Y1 = sy1_ref[0:_TOP_ROWS, :]
    X2 = sx2_ref[0:_TOP_ROWS, :]
    Y2 = sy2_ref[0:_TOP_ROWS, :]
    ar_ref[:] = (X2 - X1) * (Y2 - Y1)
    msk_ref[:] = (top_flat < _PRE_TOPK).astype(f32)

    def fill_grp(r, _):
        grp = jnp.concatenate(
            [sx1_ref[pl.ds(r, 1), :], sy1_ref[pl.ds(r, 1), :],
             sx2_ref[pl.ds(r, 1), :], sy2_ref[pl.ds(r, 1), :],
             ss_ref[pl.ds(r, 1), :],
             jnp.zeros((3, _LANES), f32)], axis=0)
        grp_ref[pl.ds(r * 8, 8), :] = grp
        return 0

    jax.lax.fori_loop(0, _TOP_ROWS, fill_grp, 0)

    def cond(carry):
        j, cnt = carry
        return (j < _TOP_ROWS * _LANES) & (cnt < _POST_TOPK)

    def body(carry):
        j, cnt = carry
        r = j >> 7
        c = j & 127
        onehot = (lane1 == c).astype(f32)

        vrow = msk_ref[pl.ds(r, 1), :]
        isv = jnp.max(jax.lax.dot_general(
            vrow, onehot, (((1,), (1,)), ((), ())),
            preferred_element_type=f32)) > 0.5

        grp = grp_ref[pl.ds(r * 8, 8), :]
        col = jax.lax.dot_general(
            grp, onehot, (((1,), (1,)), ((), ())),
            preferred_element_type=f32)

        def bc(k):
            return jnp.broadcast_to(col[k:k + 1, :], (1, _LANES))

        bx1 = bc(0)
        by1 = bc(1)
        bx2 = bc(2)
        by2 = bc(3)
        bsc = bc(4)

        row = jnp.where(
            lane1 == 0, bx1,
            jnp.where(lane1 == 1, by1,
                      jnp.where(lane1 == 2, bx2,
                                jnp.where(lane1 == 3, by2,
                                          jnp.where(lane1 == 4, bsc, -1.0)))))

        @pl.when(isv)
        def _():
            ix1 = jnp.maximum(bx1, sx1_ref[0:_TOP_ROWS, :])
            iy1 = jnp.maximum(by1, sy1_ref[0:_TOP_ROWS, :])
            ix2 = jnp.minimum(bx2, sx2_ref[0:_TOP_ROWS, :])
            iy2 = jnp.minimum(by2, sy2_ref[0:_TOP_ROWS, :])
            inter = jnp.maximum(ix2 - ix1, 0.0) * jnp.maximum(iy2 - iy1, 0.0)
            area_a = (bx2 - bx1) * (by2 - by1)
            iou = inter / (area_a + ar_ref[:] - inter + 1e-9)
            keep = (iou < _THRESH) & (top_flat != j)
            msk_ref[:] = jnp.where(keep, msk_ref[:], 0.0)
            out_ref[pl.ds(cnt, 1), :] = row

        return j + 1, cnt + isv.astype(i32)

    jax.lax.while_loop(cond, body, (jnp.int32(0), jnp.int32(0)))


def _pad2d(v, fill):
    v = jnp.concatenate([v, jnp.full((_PAD_N - _N,), fill, jnp.float32)])
    return v.reshape(_ROWS, _LANES)


def kernel(boxes, scores):
    sc = _pad2d(scores, -1.0)
    x1 = _pad2d(boxes[:, 0], 0.0)
    y1 = _pad2d(boxes[:, 1], 0.0)
    x2 = _pad2d(boxes[:, 2], 0.0)
    y2 = _pad2d(boxes[:, 3], 0.0)

    big = pltpu.VMEM((_ROWS, _LANES), jnp.float32)
    top = pltpu.VMEM((_TOP_ROWS, _LANES), jnp.float32)
    out = pl.pallas_call(
        _nms_body,
        out_shape=jax.ShapeDtypeStruct((_POST_TOPK, _LANES), jnp.float32),
        scratch_shapes=[
            big, pltpu.VMEM((_ROWS, _LANES), jnp.int32),
            big, big, big, big,
            top, top,
            pltpu.VMEM((_TOP_ROWS * 8, _LANES), jnp.float32),
        ],
    )(sc, x1, y1, x2, y2)
    return out[:, :5]


# speedup vs baseline: 4.5661x; 1.3735x over previous
import jax
import jax.numpy as jnp
from jax.experimental import pallas as pl
from jax.experimental.pallas import tpu as pltpu

_N = 20000
_PAD_N = 32768
_ROWS = 256
_LANES = 128
_TOP_ROWS = 48
_PRE_TOPK = 6000
_POST_TOPK = 1000
_THRESH = 0.7


def _nms_body(sc_in, x1_in, y1_in, x2_in, y2_in, out_ref,
              ss_ref, si_ref, sx1_ref, sy1_ref, sx2_ref, sy2_ref,
              ar_ref, msk_ref, grp_ref):
    i32 = jnp.int32
    f32 = jnp.float32
    neg_inf = jnp.float32(-jnp.inf)

    row_iota = jax.lax.broadcasted_iota(i32, (_ROWS, _LANES), 0)
    lane_iota = jax.lax.broadcasted_iota(i32, (_ROWS, _LANES), 1)
    flat_iota = row_iota * _LANES + lane_iota
    lane1 = jax.lax.broadcasted_iota(i32, (1, _LANES), 1)

    out_ref[:] = jnp.full((_POST_TOPK, _LANES), -1.0, f32)
    ss_ref[:] = sc_in[:]
    si_ref[:] = flat_iota
    sx1_ref[:] = x1_in[:]
    sy1_ref[:] = y1_in[:]
    sx2_ref[:] = x2_in[:]
    sy2_ref[:] = y2_in[:]

    def ce(d, k, axis, sp, sm):
        bit = (flat_iota & d) != 0
        dsc = (flat_iota & k) != 0

        def partner(x):
            return jnp.where(bit, pltpu.roll(x, sp, axis), pltpu.roll(x, sm, axis))

        s = ss_ref[:]
        ii = si_ref[:]
        ps = partner(s)
        pi = partner(ii)
        first = (s > ps) | ((s == ps) & (ii < pi))
        keep = first ^ bit ^ dsc
        ss_ref[:] = jnp.where(keep, s, ps)
        si_ref[:] = jnp.where(keep, ii, pi)
        for ref in (sx1_ref, sy1_ref, sx2_ref, sy2_ref):
            x = ref[:]
            ref[:] = jnp.where(keep, x, partner(x))

    def row_step(t, j):
        d = jax.lax.shift_left(jnp.int32(1), j - 1 - t)
        k = jax.lax.shift_left(jnp.int32(1), j)
        dr = d >> 7
        ce(d, k, 0, dr, (_ROWS - dr) & (_ROWS - 1))
        return j

    def lane_step(t, j):
        d = jax.lax.shift_left(jnp.int32(1), j - 1 - t)
        k = jax.lax.shift_left(jnp.int32(1), j)
        ce(d, k, 1, d, (_LANES - d) & (_LANES - 1))
        return j

    def stage(j, _):
        split = jnp.maximum(j - 7, 0)
        jax.lax.fori_loop(0, split, row_step, j)
        jax.lax.fori_loop(split, j, lane_step, j)
        return 0

    jax.lax.fori_loop(1, 16, stage, 0)

    top_flat = flat_iota[:_TOP_ROWS, :]
    lane_top = lane_iota[:_TOP_ROWS, :]

    X1 = sx1_ref[0:_TOP_ROWS, :]
    Y1 = sy1_ref[0:_TOP_ROWS, :]
    X2 = sx2_ref[0:_TOP_ROWS, :]
    Y2 = sy2_ref[0:_TOP_ROWS, :]
    ar_ref[:] = (X2 - X1) * (Y2 - Y1)
    msk_ref[:] = (top_flat < _PRE_TOPK).astype(f32)

    def fill_grp(r, _):
        grp = jnp.concatenate(
            [sx1_ref[pl.ds(r, 1), :], sy1_ref[pl.ds(r, 1), :],
             sx2_ref[pl.ds(r, 1), :], sy2_ref[pl.ds(r, 1), :],
             ss_ref[pl.ds(r, 1), :],
             jnp.zeros((3, _LANES), f32)], axis=0)
        grp_ref[pl.ds(r * 8, 8), :] = grp
        return 0

    jax.lax.fori_loop(0, _TOP_ROWS, fill_grp, 0)

    def cond(carry):
        j, cnt = carry
        return (j < _TOP_ROWS * _LANES) & (cnt < _POST_TOPK)

    def extract(j):
        r = j >> 7
        c = j & 127
        onehot = (lane1 == c).astype(f32)
        vrow = msk_ref[pl.ds(r, 1), :]
        vdot = jax.lax.dot_general(
            vrow, onehot, (((1,), (1,)), ((), ())),
            preferred_element_type=f32)
        grp = grp_ref[pl.ds(r * 8, 8), :]
        col = jax.lax.dot_general(
            grp, onehot, (((1,), (1,)), ((), ())),
            preferred_element_type=f32)

        def bc(k):
            return jnp.broadcast_to(col[k:k + 1, :], (1, _LANES))

        bx = (bc(0), bc(1), bc(2), bc(3))
        row = jnp.where(
            lane1 == 0, bx[0],
            jnp.where(lane1 == 1, bx[1],
                      jnp.where(lane1 == 2, bx[2],
                                jnp.where(lane1 == 3, bx[3],
                                          jnp.where(lane1 == 4, bc(4), -1.0)))))
        return bx, row, vdot

    def supp(bx, j):
        ix1 = jnp.maximum(bx[0], sx1_ref[0:_TOP_ROWS, :])
        iy1 = jnp.maximum(bx[1], sy1_ref[0:_TOP_ROWS, :])
        ix2 = jnp.minimum(bx[2], sx2_ref[0:_TOP_ROWS, :])
        iy2 = jnp.minimum(bx[3], sy2_ref[0:_TOP_ROWS, :])
        inter = jnp.maximum(ix2 - ix1, 0.0) * jnp.maximum(iy2 - iy1, 0.0)
        area_a = (bx[2] - bx[0]) * (bx[3] - bx[1])
        iou = inter / (area_a + ar_ref[:] - inter + 1e-9)
        return (iou < _THRESH) & (top_flat != j)

    def body(carry):
        j, cnt = carry
        j2 = jnp.minimum(j + 1, _TOP_ROWS * _LANES - 1)
        bxa, rowa, va = extract(j)
        bxb, rowb, vb = extract(j2)

        ix1 = jnp.maximum(bxa[0], bxb[0])
        iy1 = jnp.maximum(bxa[1], bxb[1])
        ix2 = jnp.minimum(bxa[2], bxb[2])
        iy2 = jnp.minimum(bxa[3], bxb[3])
        inter = jnp.maximum(ix2 - ix1, 0.0) * jnp.maximum(iy2 - iy1, 0.0)
        area_a = (bxa[2] - bxa[0]) * (bxa[3] - bxa[1])
        area_b = (bxb[2] - bxb[0]) * (bxb[3] - bxb[1])
        iou_ab = inter / (area_a + area_b - inter + 1e-9)
        ab_ok = jnp.max((iou_ab < _THRESH)[:, 0:1].astype(f32))

        isva = jnp.max(va) > 0.5
        isvb_raw = jnp.max(vb) > 0.5
        isvb = isvb_raw & (j + 1 < _TOP_ROWS * _LANES) & \
            ((~isva) | (ab_ok > 0.5))

        keepa = supp(bxa, j)
        keepb = supp(bxb, j2)
        ka = keepa | (~isva)
        kb = keepb | (~isvb)
        msk_ref[:] = jnp.where(ka & kb, msk_ref[:], 0.0)

        na = isva.astype(i32)
        nb = isvb.astype(i32)

        @pl.when(isva)
        def _():
            out_ref[pl.ds(cnt, 1), :] = rowa

        @pl.when(isvb & (cnt + na < _POST_TOPK))
        def _():
            out_ref[pl.ds(cnt + na, 1), :] = rowb

        return j + 2, cnt + na + nb

    jax.lax.while_loop(cond, body, (jnp.int32(0), jnp.int32(0)))


def _pad2d(v, fill):
    v = jnp.concatenate([v, jnp.full((_PAD_N - _N,), fill, jnp.float32)])
    return v.reshape(_ROWS, _LANES)


def kernel(boxes, scores):
    sc = _pad2d(scores, -1.0)
    x1 = _pad2d(boxes[:, 0], 0.0)
    y1 = _pad2d(boxes[:, 1], 0.0)
    x2 = _pad2d(boxes[:, 2], 0.0)
    y2 = _pad2d(boxes[:, 3], 0.0)

    big = pltpu.VMEM((_ROWS, _LANES), jnp.float32)
    top = pltpu.VMEM((_TOP_ROWS, _LANES), jnp.float32)
    out = pl.pallas_call(
        _nms_body,
        out_shape=jax.ShapeDtypeStruct((_POST_TOPK, _LANES), jnp.float32),
        scratch_shapes=[
            big, pltpu.VMEM((_ROWS, _LANES), jnp.int32),
            big, big, big, big,
            top, top,
            pltpu.VMEM((_TOP_ROWS * 8, _LANES), jnp.float32),
        ],
    )(sc, x1, y1, x2, y2)
    return out[:, :5]
